# Initial kernel scaffold; baseline (speedup 1.0000x reference)
import jax
import jax.numpy as jnp
from jax.experimental import pallas as pl


def kernel(pred_logits, pred_masks):
    raise NotImplementedError("write your pallas kernel here")


# speedup vs baseline: 1.1478x; 1.1478x over previous
import jax
import jax.numpy as jnp
from jax import lax
from jax.experimental import pallas as pl

_MASK_CONST = -99999.0
_THRESHOLD = 0.1

Q = 1000
C = 21
P = 128 * 128
BLK = 2048
GRID = P // BLK


def _body(logits_ref, masks_ref, ms_ref, cls_ref, weighted_ref, mid_ref):
    logits = logits_ref[...]
    lmax = jnp.max(logits, axis=1, keepdims=True)
    e = jnp.exp(logits - lmax)
    s = jnp.sum(e, axis=1, keepdims=True)
    probs = e / s
    pmax = jnp.max(probs, axis=1, keepdims=True)
    citer = lax.broadcasted_iota(jnp.int32, (Q, C), 1)
    cls = jnp.min(jnp.where(probs == pmax, citer, jnp.int32(2**30)),
                  axis=1, keepdims=True)
    scores = pmax
    keep = (cls != (C - 1)) & (scores > _THRESHOLD)
    ms = scores * keep.astype(scores.dtype)
    ms_ref[...] = ms
    cls_ref[...] = cls

    x = masks_ref[...]
    xm = jnp.where(keep, x, _MASK_CONST)
    mx = jnp.max(xm, axis=0, keepdims=True)
    ex = jnp.exp(xm - mx)
    denom = jnp.sum(ex, axis=0, keepdims=True)
    soft = ex / denom
    weighted_ref[...] = soft * ms
    smax = jnp.max(soft, axis=0, keepdims=True)
    qiter = lax.broadcasted_iota(jnp.int32, (Q, BLK), 0)
    mid_ref[...] = jnp.min(
        jnp.where(soft == smax, qiter, jnp.int32(2**30)), axis=0, keepdims=True)


def kernel(pred_logits, pred_masks):
    flat = pred_masks.reshape(Q, P)
    ms, cls, weighted, m_id = pl.pallas_call(
        _body,
        grid=(GRID,),
        in_specs=[
            pl.BlockSpec((Q, C), lambda i: (0, 0)),
            pl.BlockSpec((Q, BLK), lambda i: (0, i)),
        ],
        out_specs=[
            pl.BlockSpec((Q, 1), lambda i: (0, 0)),
            pl.BlockSpec((Q, 1), lambda i: (0, 0)),
            pl.BlockSpec((Q, BLK), lambda i: (0, i)),
            pl.BlockSpec((1, BLK), lambda i: (0, i)),
        ],
        out_shape=[
            jax.ShapeDtypeStruct((Q, 1), jnp.float32),
            jax.ShapeDtypeStruct((Q, 1), jnp.int32),
            jax.ShapeDtypeStruct((Q, P), jnp.float32),
            jax.ShapeDtypeStruct((1, P), jnp.int32),
        ],
    )(pred_logits, flat)
    return (ms.reshape(Q), cls.reshape(Q), weighted, m_id.reshape(P))


# speedup vs baseline: 2.1657x; 1.8868x over previous
import jax
import jax.numpy as jnp
from jax import lax
from jax.experimental import pallas as pl

_MASK_CONST = -99999.0
_THRESHOLD = 0.1

Q = 1000
C = 21
P = 128 * 128
BLK = 2048
HB = BLK // 128
GRID = P // BLK


def _body(logits_ref, masks_ref, ms_ref, cls_ref, weighted_ref, mid_ref):
    logits = logits_ref[...]
    lmax = jnp.max(logits, axis=1, keepdims=True)
    e = jnp.exp(logits - lmax)
    s = jnp.sum(e, axis=1, keepdims=True)
    probs = e / s
    pmax = jnp.max(probs, axis=1, keepdims=True)
    citer = lax.broadcasted_iota(jnp.int32, (Q, C), 1)
    cls = jnp.min(jnp.where(probs == pmax, citer, jnp.int32(2**30)),
                  axis=1, keepdims=True)
    scores = pmax
    keep = (cls != (C - 1)) & (scores > _THRESHOLD)
    ms = scores * keep.astype(scores.dtype)
    ms_ref[...] = ms
    cls_ref[...] = cls

    x = masks_ref[...].reshape(Q, BLK)
    xm = jnp.where(keep, x, _MASK_CONST)
    mx = jnp.max(xm, axis=0, keepdims=True)
    ex = jnp.exp(xm - mx)
    denom = jnp.sum(ex, axis=0, keepdims=True)
    soft = ex / denom
    weighted_ref[...] = soft * ms
    smax = 1.0 / denom
    qiter = lax.broadcasted_iota(jnp.int32, (Q, BLK), 0)
    mid_ref[...] = jnp.min(
        jnp.where(soft == smax, qiter, jnp.int32(2**30)), axis=0, keepdims=True)


def kernel(pred_logits, pred_masks):
    ms, cls, weighted, m_id = pl.pallas_call(
        _body,
        grid=(GRID,),
        in_specs=[
            pl.BlockSpec((Q, C), lambda i: (0, 0)),
            pl.BlockSpec((Q, HB, 128), lambda i: (0, i, 0)),
        ],
        out_specs=[
            pl.BlockSpec((Q, 1), lambda i: (0, 0)),
            pl.BlockSpec((Q, 1), lambda i: (0, 0)),
            pl.BlockSpec((Q, BLK), lambda i: (0, i)),
            pl.BlockSpec((1, BLK), lambda i: (0, i)),
        ],
        out_shape=[
            jax.ShapeDtypeStruct((Q, 1), jnp.float32),
            jax.ShapeDtypeStruct((Q, 1), jnp.int32),
            jax.ShapeDtypeStruct((Q, P), jnp.float32),
            jax.ShapeDtypeStruct((1, P), jnp.int32),
        ],
    )(pred_logits, pred_masks)
    return (ms.reshape(Q), cls.reshape(Q), weighted, m_id.reshape(P))


# speedup vs baseline: 2.2422x; 1.0353x over previous
import jax
import jax.numpy as jnp
from jax import lax
from jax.experimental import pallas as pl

_MASK_CONST = -99999.0
_THRESHOLD = 0.1

Q = 1000
C = 21
P = 128 * 128
BLK = 2048
HB = BLK // 128
GRID = P // BLK


def _body(logits_ref, masks_ref, ms_ref, cls_ref, weighted_ref, mid_ref):
    logits = logits_ref[...]
    lmax = jnp.max(logits, axis=1, keepdims=True)
    e = jnp.exp(logits - lmax)
    s = jnp.sum(e, axis=1, keepdims=True)
    probs = e / s
    pmax = jnp.max(probs, axis=1, keepdims=True)
    citer = lax.broadcasted_iota(jnp.int32, (Q, C), 1)
    cls = jnp.min(jnp.where(probs == pmax, citer, jnp.int32(2**30)),
                  axis=1, keepdims=True)
    scores = pmax
    keep = (cls != (C - 1)) & (scores > _THRESHOLD)
    ms = scores * keep.astype(scores.dtype)
    ms_ref[...] = ms
    cls_ref[...] = cls

    x = masks_ref[...].reshape(Q, BLK)
    xm = jnp.where(keep, x, _MASK_CONST)
    mx = jnp.max(xm, axis=0, keepdims=True)
    ex = jnp.exp(xm - mx)
    denom = jnp.sum(ex, axis=0, keepdims=True)
    soft = ex / denom
    weighted_ref[...] = soft * ms
    mid_ref[...] = jnp.argmax(soft, axis=0, keepdims=True)


def kernel(pred_logits, pred_masks):
    ms, cls, weighted, m_id = pl.pallas_call(
        _body,
        grid=(GRID,),
        in_specs=[
            pl.BlockSpec((Q, C), lambda i: (0, 0)),
            pl.BlockSpec((Q, HB, 128), lambda i: (0, i, 0)),
        ],
        out_specs=[
            pl.BlockSpec((Q, 1), lambda i: (0, 0)),
            pl.BlockSpec((Q, 1), lambda i: (0, 0)),
            pl.BlockSpec((Q, BLK), lambda i: (0, i)),
            pl.BlockSpec((1, BLK), lambda i: (0, i)),
        ],
        out_shape=[
            jax.ShapeDtypeStruct((Q, 1), jnp.float32),
            jax.ShapeDtypeStruct((Q, 1), jnp.int32),
            jax.ShapeDtypeStruct((Q, P), jnp.float32),
            jax.ShapeDtypeStruct((1, P), jnp.int32),
        ],
    )(pred_logits, pred_masks)
    return (ms.reshape(Q), cls.reshape(Q), weighted, m_id.reshape(P))


# speedup vs baseline: 2.2448x; 1.0011x over previous
import jax
import jax.numpy as jnp
from jax import lax
from jax.experimental import pallas as pl

_MASK_CONST = -99999.0
_THRESHOLD = 0.1

Q = 1000
C = 21
P = 128 * 128
BLK = 2048
HB = BLK // 128
GRID = P // BLK


def _body(logits_ref, masks_ref, ms_ref, cls_ref, weighted_ref, mid_ref):
    @pl.when(pl.program_id(0) == 0)
    def _():
        logits = logits_ref[...]
        lmax = jnp.max(logits, axis=1, keepdims=True)
        e = jnp.exp(logits - lmax)
        s = jnp.sum(e, axis=1, keepdims=True)
        probs = e / s
        pmax = jnp.max(probs, axis=1, keepdims=True)
        citer = lax.broadcasted_iota(jnp.int32, (Q, C), 1)
        cls = jnp.min(jnp.where(probs == pmax, citer, jnp.int32(2**30)),
                      axis=1, keepdims=True)
        kp = (cls != (C - 1)) & (pmax > _THRESHOLD)
        ms_ref[...] = pmax * kp.astype(jnp.float32)
        cls_ref[...] = cls

    ms = ms_ref[...]
    keep = ms > 0.0

    x = masks_ref[...].reshape(Q, BLK)
    xm = jnp.where(keep, x, _MASK_CONST)
    mx = jnp.max(xm, axis=0, keepdims=True)
    ex = jnp.exp(xm - mx)
    denom = jnp.sum(ex, axis=0, keepdims=True)
    soft = ex / denom
    weighted_ref[...] = soft * ms
    mid_ref[...] = jnp.argmax(soft, axis=0, keepdims=True)


def kernel(pred_logits, pred_masks):
    ms, cls, weighted, m_id = pl.pallas_call(
        _body,
        grid=(GRID,),
        in_specs=[
            pl.BlockSpec((Q, C), lambda i: (0, 0)),
            pl.BlockSpec((Q, HB, 128), lambda i: (0, i, 0)),
        ],
        out_specs=[
            pl.BlockSpec((Q, 1), lambda i: (0, 0)),
            pl.BlockSpec((Q, 1), lambda i: (0, 0)),
            pl.BlockSpec((Q, BLK), lambda i: (0, i)),
            pl.BlockSpec((1, BLK), lambda i: (0, i)),
        ],
        out_shape=[
            jax.ShapeDtypeStruct((Q, 1), jnp.float32),
            jax.ShapeDtypeStruct((Q, 1), jnp.int32),
            jax.ShapeDtypeStruct((Q, P), jnp.float32),
            jax.ShapeDtypeStruct((1, P), jnp.int32),
        ],
    )(pred_logits, pred_masks)
    return (ms.reshape(Q), cls.reshape(Q), weighted, m_id.reshape(P))


# speedup vs baseline: 2.2659x; 1.0094x over previous
import jax
import jax.numpy as jnp
from jax import lax
from jax.experimental import pallas as pl

_MASK_CONST = -99999.0
_THRESHOLD = 0.1

Q = 1000
C = 21
P = 128 * 128
BLK = 1024
HB = BLK // 128
GRID = P // BLK


def _body(logits_ref, masks_ref, ms_ref, cls_ref, weighted_ref, mid_ref):
    @pl.when(pl.program_id(0) == 0)
    def _():
        logits = logits_ref[...]
        lmax = jnp.max(logits, axis=1, keepdims=True)
        e = jnp.exp(logits - lmax)
        s = jnp.sum(e, axis=1, keepdims=True)
        probs = e / s
        pmax = jnp.max(probs, axis=1, keepdims=True)
        citer = lax.broadcasted_iota(jnp.int32, (Q, C), 1)
        cls = jnp.min(jnp.where(probs == pmax, citer, jnp.int32(2**30)),
                      axis=1, keepdims=True)
        kp = (cls != (C - 1)) & (pmax > _THRESHOLD)
        ms_ref[...] = pmax * kp.astype(jnp.float32)
        cls_ref[...] = cls

    ms = ms_ref[...]
    keep = ms > 0.0

    x = masks_ref[...].reshape(Q, BLK)
    xm = jnp.where(keep, x, _MASK_CONST)
    mx = jnp.max(xm, axis=0, keepdims=True)
    ex = jnp.exp(xm - mx)
    denom = jnp.sum(ex, axis=0, keepdims=True)
    soft = ex / denom
    weighted_ref[...] = soft * ms
    mid_ref[...] = jnp.argmax(soft, axis=0, keepdims=True)


def kernel(pred_logits, pred_masks):
    ms, cls, weighted, m_id = pl.pallas_call(
        _body,
        grid=(GRID,),
        in_specs=[
            pl.BlockSpec((Q, C), lambda i: (0, 0)),
            pl.BlockSpec((Q, HB, 128), lambda i: (0, i, 0)),
        ],
        out_specs=[
            pl.BlockSpec((Q, 1), lambda i: (0, 0)),
            pl.BlockSpec((Q, 1), lambda i: (0, 0)),
            pl.BlockSpec((Q, BLK), lambda i: (0, i)),
            pl.BlockSpec((1, BLK), lambda i: (0, i)),
        ],
        out_shape=[
            jax.ShapeDtypeStruct((Q, 1), jnp.float32),
            jax.ShapeDtypeStruct((Q, 1), jnp.int32),
            jax.ShapeDtypeStruct((Q, P), jnp.float32),
            jax.ShapeDtypeStruct((1, P), jnp.int32),
        ],
    )(pred_logits, pred_masks)
    return (ms.reshape(Q), cls.reshape(Q), weighted, m_id.reshape(P))
